# Initial kernel scaffold; baseline (speedup 1.0000x reference)
import jax
import jax.numpy as jnp
from jax.experimental import pallas as pl


def kernel(x, edge_index, edge_attr, batch, mp_W1, mp_b1, mp_W2, mp_b2, mp_root, mp_bias, dmp_W1, dmp_b1, dmp_W2, dmp_b2, dmp_root, dmp_bias, gru_w_ih, gru_w_hh, gru_b_ih, gru_b_hh, lin_W, lin_b, bn_gamma, bn_beta, sc_W, sc_b):
    raise NotImplementedError("write your pallas kernel here")


# speedup vs baseline: 1.1242x; 1.1242x over previous
import functools

import jax
import jax.numpy as jnp
from jax import lax
from jax.experimental import pallas as pl
from jax.experimental.pallas import tpu as pltpu
from jax.experimental.pallas import tpu_sc as plsc

N = 10000
E = 160000
D = 16
DE = 16
HID = 64
DOUT = 64

NC = 2
NS = 16
NW = NC * NS
EPT = E // NW
CH = 125
NCHUNK = EPT // CH
N_PAD = 10240
RPT = N_PAD // NS

_mesh = plsc.VectorSubcoreMesh(
    core_axis_name="c", subcore_axis_name="s", num_cores=NC, num_subcores=NS)

_sc_params = pltpu.CompilerParams(use_tc_tiling_on_sc=False)

f32 = jnp.float32


@functools.partial(
    pl.kernel,
    out_type=jax.ShapeDtypeStruct((E, D), f32),
    mesh=_mesh,
    scratch_types=[
        pltpu.VMEM((NCHUNK, CH), jnp.int32),
        pltpu.VMEM((EPT, D), f32),
        pltpu.SemaphoreType.DMA,
    ],
    compiler_params=_sc_params,
)
def _sc_gather(table_hbm, idx2_hbm, out_hbm, idx_v, rows_v, sem):
    wid = lax.axis_index("s") * NC + lax.axis_index("c")
    pltpu.sync_copy(idx2_hbm.at[pl.ds(wid * NCHUNK, NCHUNK)], idx_v)
    base = wid * EPT

    def body(j, carry):
        pltpu.async_copy(
            table_hbm.at[idx_v.at[j]], rows_v.at[pl.ds(j * CH, CH)], sem
        ).wait()
        return carry

    lax.fori_loop(0, NCHUNK, body, 0, unroll=False)
    pltpu.sync_copy(rows_v, out_hbm.at[pl.ds(base, EPT)])


@functools.partial(
    pl.kernel,
    out_type=jax.ShapeDtypeStruct((NC, N_PAD, D), f32),
    mesh=_mesh,
    scratch_types=[
        pltpu.VMEM((NCHUNK, CH), jnp.int32),
        pltpu.VMEM((EPT, D), f32),
        pltpu.VMEM_SHARED((N_PAD, D), f32),
        pltpu.SemaphoreType.DMA,
    ],
    compiler_params=_sc_params,
)
def _sc_scatter(msg_hbm, idx2_hbm, out_hbm, idx_v, msg_v, acc_sh, sem):
    cid = lax.axis_index("c")
    sid = lax.axis_index("s")
    wid = sid * NC + cid
    pltpu.sync_copy(idx2_hbm.at[pl.ds(wid * NCHUNK, NCHUNK)], idx_v)

    def zbody(i, carry):
        msg_v[i, :] = jnp.zeros((D,), f32)
        return carry

    lax.fori_loop(0, 128, zbody, 0, unroll=False)

    def zcopy(i, carry):
        pltpu.sync_copy(msg_v.at[pl.ds(0, 128)],
                        acc_sh.at[pl.ds(sid * RPT + i * 128, 128)])
        return carry

    lax.fori_loop(0, RPT // 128, zcopy, 0, unroll=False)
    plsc.subcore_barrier()

    base = wid * EPT
    pltpu.sync_copy(msg_hbm.at[pl.ds(base, EPT)], msg_v)

    def body(j, carry):
        pltpu.sync_copy(msg_v.at[pl.ds(j * CH, CH)],
                        acc_sh.at[idx_v.at[j]], add=True)
        return carry

    lax.fori_loop(0, NCHUNK, body, 0, unroll=False)
    plsc.subcore_barrier()
    pltpu.sync_copy(acc_sh.at[pl.ds(sid * RPT, RPT)],
                    out_hbm.at[cid, pl.ds(sid * RPT, RPT)])


BE = 1600
GE = E // BE


def _msg_body(ea_ref, xs_ref, ic_ref, W1_ref, b1_ref, W2_ref, b2_ref, out_ref):
    h = jnp.maximum(
        jnp.dot(ea_ref[...], W1_ref[...], preferred_element_type=f32)
        + b1_ref[...], 0.0)
    w = jnp.dot(h, W2_ref[...], preferred_element_type=f32) + b2_ref[...]
    xs = xs_ref[...]
    acc = xs[:, 0:1] * w[:, 0:D]
    for i in range(1, D):
        acc = acc + xs[:, i:i + 1] * w[:, i * D:(i + 1) * D]
    out_ref[...] = acc * ic_ref[...]


_msg_call = pl.pallas_call(
    _msg_body,
    grid=(GE,),
    in_specs=[
        pl.BlockSpec((BE, DE), lambda i: (i, 0)),
        pl.BlockSpec((BE, D), lambda i: (i, 0)),
        pl.BlockSpec((BE, D), lambda i: (i, 0)),
        pl.BlockSpec((DE, HID), lambda i: (0, 0)),
        pl.BlockSpec((1, HID), lambda i: (0, 0)),
        pl.BlockSpec((HID, D * D), lambda i: (0, 0)),
        pl.BlockSpec((1, D * D), lambda i: (0, 0)),
    ],
    out_specs=pl.BlockSpec((BE, D), lambda i: (i, 0)),
    out_shape=jax.ShapeDtypeStruct((E, D), f32),
)


def _inv_body(p_ref, out_ref):
    c = p_ref[0] + p_ref[1]
    out_ref[...] = 1.0 / jnp.maximum(c, 1.0)


_inv_call = pl.pallas_call(
    _inv_body,
    out_shape=jax.ShapeDtypeStruct((N, D), f32),
)


def _combine_body(p_ref, cur_ref, root_ref, bias_ref, out_ref):
    aggr = p_ref[0] + p_ref[1]
    out_ref[...] = aggr + jnp.dot(
        cur_ref[...], root_ref[...], preferred_element_type=f32) + bias_ref[...]


_combine_call = pl.pallas_call(
    _combine_body,
    out_shape=jax.ShapeDtypeStruct((N, D), f32),
)


def _gru_body(p_ref, m1_ref, h_ref, root_ref, bias_ref, wihT_ref, whhT_ref,
              bih_ref, bhh_ref, out_ref):
    m1 = m1_ref[...]
    h = h_ref[...]
    m2 = (p_ref[0] + p_ref[1]
          + jnp.dot(m1, root_ref[...], preferred_element_type=f32)
          + bias_ref[...])
    gi = jnp.dot(m2, wihT_ref[...], preferred_element_type=f32) + bih_ref[...]
    gh = jnp.dot(h, whhT_ref[...], preferred_element_type=f32) + bhh_ref[...]
    r = jax.nn.sigmoid(gi[:, :D] + gh[:, :D])
    z = jax.nn.sigmoid(gi[:, D:2 * D] + gh[:, D:2 * D])
    n_ = jnp.tanh(gi[:, 2 * D:] + r * gh[:, 2 * D:])
    out_ref[...] = (1.0 - z) * n_ + z * h


_gru_call = pl.pallas_call(
    _gru_body,
    out_shape=jax.ShapeDtypeStruct((N, D), f32),
)


def _head_body(cur_ref, linW_ref, linb_ref, g_ref, b_ref, scW_ref, scb_ref,
               out_ref):
    cur = cur_ref[...]
    y = jnp.dot(cur, linW_ref[...], preferred_element_type=f32) + linb_ref[...]
    mean = jnp.mean(y, axis=0, keepdims=True)
    var = jnp.mean((y - mean) ** 2, axis=0, keepdims=True)
    yn = (y - mean) / jnp.sqrt(var + 1e-5) * g_ref[...] + b_ref[...]
    out_ref[...] = jnp.maximum(yn, 0.0) + jnp.dot(
        cur, scW_ref[...], preferred_element_type=f32) + scb_ref[...]


_head_call = pl.pallas_call(
    _head_body,
    out_shape=jax.ShapeDtypeStruct((N, DOUT), f32),
)


def kernel(x, edge_index, edge_attr, batch, mp_W1, mp_b1, mp_W2, mp_b2,
           mp_root, mp_bias, dmp_W1, dmp_b1, dmp_W2, dmp_b2, dmp_root,
           dmp_bias, gru_w_ih, gru_w_hh, gru_b_ih, gru_b_hh, lin_W, lin_b,
           bn_gamma, bn_beta, sc_W, sc_b):
    src2 = edge_index[0].reshape(E // CH, CH)
    dst2 = edge_index[1].reshape(E // CH, CH)

    mp_b1r = mp_b1.reshape(1, HID)
    mp_b2r = mp_b2.reshape(1, D * D)
    mp_biasr = mp_bias.reshape(1, D)
    dmp_b1r = dmp_b1.reshape(1, HID)
    dmp_b2r = dmp_b2.reshape(1, D * D)
    dmp_biasr = dmp_bias.reshape(1, D)
    wihT = gru_w_ih.T
    whhT = gru_w_hh.T
    bihr = gru_b_ih.reshape(1, 3 * D)
    bhhr = gru_b_hh.reshape(1, 3 * D)
    linbr = lin_b.reshape(1, DOUT)
    gammar = bn_gamma.reshape(1, DOUT)
    betar = bn_beta.reshape(1, DOUT)
    scbr = sc_b.reshape(1, DOUT)

    ones_e = jnp.ones((E, D), f32)
    cnt_p = _sc_scatter(ones_e, dst2)[:, :N]
    inv = _inv_call(cnt_p)
    icnt_e = _sc_gather(inv, dst2)

    def nnconv_partials(cur, W1, b1, W2, b2):
        xs = _sc_gather(cur, src2)
        msg = _msg_call(edge_attr, xs, icnt_e, W1, b1, W2, b2)
        return _sc_scatter(msg, dst2)[:, :N]

    h = x
    cur = x
    for _ in range(3):
        p1 = nnconv_partials(cur, mp_W1, mp_b1r, mp_W2, mp_b2r)
        m1 = _combine_call(p1, cur, mp_root, mp_biasr)
        p2 = nnconv_partials(m1, dmp_W1, dmp_b1r, dmp_W2, dmp_b2r)
        h = _gru_call(p2, m1, h, dmp_root, dmp_biasr, wihT, whhT, bihr, bhhr)
        cur = h

    return _head_call(cur, lin_W, linbr, gammar, betar, sc_W, scbr)


# speedup vs baseline: 2.9382x; 2.6135x over previous
import functools

import jax
import jax.numpy as jnp
from jax import lax
from jax.experimental import pallas as pl
from jax.experimental.pallas import tpu as pltpu
from jax.experimental.pallas import tpu_sc as plsc

N = 10000
E = 160000
D = 16
DE = 16
HID = 64
DOUT = 64

NC = 2
NS = 16
NW = NC * NS
EPT = E // NW
CH = 125
NCHUNK = EPT // CH
N_PAD = 10240
RPT = N_PAD // NS

CHF = 128
BASE_CH = 39
EPT2 = BASE_CH * CHF
EXTRA = 2
MAXCH = BASE_CH + EXTRA

_mesh = plsc.VectorSubcoreMesh(
    core_axis_name="c", subcore_axis_name="s", num_cores=NC, num_subcores=NS)

_sc_params = pltpu.CompilerParams(use_tc_tiling_on_sc=False)

f32 = jnp.float32


@functools.partial(
    pl.kernel,
    out_type=jax.ShapeDtypeStruct((NC, N_PAD, D), f32),
    mesh=_mesh,
    scratch_types=[
        pltpu.VMEM((NCHUNK, CH), jnp.int32),
        pltpu.VMEM((EPT, D), f32),
        pltpu.VMEM_SHARED((N_PAD, D), f32),
        pltpu.SemaphoreType.DMA,
    ],
    compiler_params=_sc_params,
)
def _sc_scatter(msg_hbm, idx2_hbm, out_hbm, idx_v, msg_v, acc_sh, sem):
    cid = lax.axis_index("c")
    sid = lax.axis_index("s")
    wid = sid * NC + cid
    pltpu.sync_copy(idx2_hbm.at[pl.ds(wid * NCHUNK, NCHUNK)], idx_v)

    def zbody(i, carry):
        msg_v[i, :] = jnp.zeros((D,), f32)
        return carry

    lax.fori_loop(0, 128, zbody, 0, unroll=False)

    def zcopy(i, carry):
        pltpu.sync_copy(msg_v.at[pl.ds(0, 128)],
                        acc_sh.at[pl.ds(sid * RPT + i * 128, 128)])
        return carry

    lax.fori_loop(0, RPT // 128, zcopy, 0, unroll=False)
    plsc.subcore_barrier()

    base = wid * EPT
    pltpu.sync_copy(msg_hbm.at[pl.ds(base, EPT)], msg_v)

    def body(j, carry):
        pltpu.sync_copy(msg_v.at[pl.ds(j * CH, CH)],
                        acc_sh.at[idx_v.at[j]], add=True)
        return carry

    lax.fori_loop(0, NCHUNK, body, 0, unroll=False)
    plsc.subcore_barrier()
    pltpu.sync_copy(acc_sh.at[pl.ds(sid * RPT, RPT)],
                    out_hbm.at[cid, pl.ds(sid * RPT, RPT)])


@functools.partial(
    pl.kernel,
    out_type=jax.ShapeDtypeStruct((NC, N_PAD, D), f32),
    mesh=_mesh,
    scratch_types=[
        pltpu.VMEM((MAXCH, CHF), jnp.int32),
        pltpu.VMEM((MAXCH, CHF), jnp.int32),
        pltpu.VMEM((CHF, D), f32),
        pltpu.VMEM((CHF, 8 * D), f32),
        pltpu.VMEM((CHF, 8 * D), f32),
        pltpu.VMEM((CHF, D), f32),
        pltpu.VMEM_SHARED((N_PAD, D), f32),
        pltpu.SemaphoreType.DMA,
    ],
    compiler_params=_sc_params,
)
def _sc_pass(table_hbm, src3_hbm, dst3_hbm, wa_hbm, wb_hbm, out_hbm,
             sidx_v, didx_v, xs_v, wa_v, wb_v, msg_v, acc_sh, sem):
    cid = lax.axis_index("c")
    sid = lax.axis_index("s")
    wid = sid * NC + cid
    pltpu.sync_copy(src3_hbm.at[wid], sidx_v)
    pltpu.sync_copy(dst3_hbm.at[wid], didx_v)

    def zbody(i, carry):
        msg_v[i, :] = jnp.zeros((D,), f32)
        return carry

    lax.fori_loop(0, CHF, zbody, 0, unroll=False)

    def zcopy(i, carry):
        pltpu.sync_copy(msg_v, acc_sh.at[pl.ds(sid * RPT + i * CHF, CHF)])
        return carry

    lax.fori_loop(0, RPT // CHF, zcopy, 0, unroll=False)
    plsc.subcore_barrier()

    nch = jnp.where(wid == 0, MAXCH, BASE_CH)

    def body(j, carry):
        eoff = jnp.where(j < BASE_CH,
                         wid * EPT2 + j * CHF,
                         NW * EPT2 + (j - BASE_CH) * CHF)
        pltpu.async_copy(wa_hbm.at[pl.ds(eoff, CHF)], wa_v, sem).wait()
        pltpu.async_copy(wb_hbm.at[pl.ds(eoff, CHF)], wb_v, sem).wait()
        pltpu.async_copy(table_hbm.at[sidx_v.at[j]], xs_v, sem).wait()

        def edge(e, carry2):
            xsrow = xs_v[e, :]
            acc = xsrow[0] * wa_v[e, 0:D]
            for i in range(1, 8):
                acc = acc + xsrow[i] * wa_v[e, i * D:(i + 1) * D]
            for i in range(8):
                acc = acc + xsrow[8 + i] * wb_v[e, i * D:(i + 1) * D]
            msg_v[e, :] = acc
            return carry2

        lax.fori_loop(0, CHF, edge, 0, unroll=False)
        pltpu.sync_copy(msg_v, acc_sh.at[didx_v.at[j]], add=True)
        return carry

    lax.fori_loop(0, nch, body, 0, unroll=False)
    plsc.subcore_barrier()
    pltpu.sync_copy(acc_sh.at[pl.ds(sid * RPT, RPT)],
                    out_hbm.at[cid, pl.ds(sid * RPT, RPT)])


BE = 1600
GE = E // BE


def _wprep_body(ea_ref, W1m_ref, b1m_ref, W2m_ref, b2m_ref,
                W1d_ref, b1d_ref, W2d_ref, b2d_ref,
                wam_ref, wbm_ref, wad_ref, wbd_ref):
    ea = ea_ref[...]
    hm = jnp.maximum(
        jnp.dot(ea, W1m_ref[...], preferred_element_type=f32)
        + b1m_ref[...], 0.0)
    wm = jnp.dot(hm, W2m_ref[...], preferred_element_type=f32) + b2m_ref[...]
    wam_ref[...] = wm[:, :8 * D]
    wbm_ref[...] = wm[:, 8 * D:]
    hd = jnp.maximum(
        jnp.dot(ea, W1d_ref[...], preferred_element_type=f32)
        + b1d_ref[...], 0.0)
    wd = jnp.dot(hd, W2d_ref[...], preferred_element_type=f32) + b2d_ref[...]
    wad_ref[...] = wd[:, :8 * D]
    wbd_ref[...] = wd[:, 8 * D:]


_wprep_call = pl.pallas_call(
    _wprep_body,
    grid=(GE,),
    in_specs=[
        pl.BlockSpec((BE, DE), lambda i: (i, 0)),
        pl.BlockSpec((DE, HID), lambda i: (0, 0)),
        pl.BlockSpec((1, HID), lambda i: (0, 0)),
        pl.BlockSpec((HID, D * D), lambda i: (0, 0)),
        pl.BlockSpec((1, D * D), lambda i: (0, 0)),
        pl.BlockSpec((DE, HID), lambda i: (0, 0)),
        pl.BlockSpec((1, HID), lambda i: (0, 0)),
        pl.BlockSpec((HID, D * D), lambda i: (0, 0)),
        pl.BlockSpec((1, D * D), lambda i: (0, 0)),
    ],
    out_specs=[
        pl.BlockSpec((BE, 8 * D), lambda i: (i, 0)),
        pl.BlockSpec((BE, 8 * D), lambda i: (i, 0)),
        pl.BlockSpec((BE, 8 * D), lambda i: (i, 0)),
        pl.BlockSpec((BE, 8 * D), lambda i: (i, 0)),
    ],
    out_shape=[
        jax.ShapeDtypeStruct((E, 8 * D), f32),
        jax.ShapeDtypeStruct((E, 8 * D), f32),
        jax.ShapeDtypeStruct((E, 8 * D), f32),
        jax.ShapeDtypeStruct((E, 8 * D), f32),
    ],
)


def _inv_body(p_ref, out_ref):
    c = p_ref[0] + p_ref[1]
    out_ref[...] = 1.0 / jnp.maximum(c, 1.0)


_inv_call = pl.pallas_call(
    _inv_body,
    out_shape=jax.ShapeDtypeStruct((N, D), f32),
)


def _combine_body(p_ref, inv_ref, cur_ref, root_ref, bias_ref, out_ref):
    aggr = (p_ref[0] + p_ref[1]) * inv_ref[...]
    out_ref[...] = aggr + jnp.dot(
        cur_ref[...], root_ref[...], preferred_element_type=f32) + bias_ref[...]


_combine_call = pl.pallas_call(
    _combine_body,
    out_shape=jax.ShapeDtypeStruct((N, D), f32),
)


def _gru_body(p_ref, inv_ref, m1_ref, h_ref, root_ref, bias_ref, wihT_ref,
              whhT_ref, bih_ref, bhh_ref, out_ref):
    m1 = m1_ref[...]
    h = h_ref[...]
    m2 = ((p_ref[0] + p_ref[1]) * inv_ref[...]
          + jnp.dot(m1, root_ref[...], preferred_element_type=f32)
          + bias_ref[...])
    gi = jnp.dot(m2, wihT_ref[...], preferred_element_type=f32) + bih_ref[...]
    gh = jnp.dot(h, whhT_ref[...], preferred_element_type=f32) + bhh_ref[...]
    r = jax.nn.sigmoid(gi[:, :D] + gh[:, :D])
    z = jax.nn.sigmoid(gi[:, D:2 * D] + gh[:, D:2 * D])
    n_ = jnp.tanh(gi[:, 2 * D:] + r * gh[:, 2 * D:])
    out_ref[...] = (1.0 - z) * n_ + z * h


_gru_call = pl.pallas_call(
    _gru_body,
    out_shape=jax.ShapeDtypeStruct((N, D), f32),
)


def _head_body(cur_ref, linW_ref, linb_ref, g_ref, b_ref, scW_ref, scb_ref,
               out_ref):
    cur = cur_ref[...]
    y = jnp.dot(cur, linW_ref[...], preferred_element_type=f32) + linb_ref[...]
    mean = jnp.mean(y, axis=0, keepdims=True)
    var = jnp.mean((y - mean) ** 2, axis=0, keepdims=True)
    yn = (y - mean) / jnp.sqrt(var + 1e-5) * g_ref[...] + b_ref[...]
    out_ref[...] = jnp.maximum(yn, 0.0) + jnp.dot(
        cur, scW_ref[...], preferred_element_type=f32) + scb_ref[...]


_head_call = pl.pallas_call(
    _head_body,
    out_shape=jax.ShapeDtypeStruct((N, DOUT), f32),
)


def kernel(x, edge_index, edge_attr, batch, mp_W1, mp_b1, mp_W2, mp_b2,
           mp_root, mp_bias, dmp_W1, dmp_b1, dmp_W2, dmp_b2, dmp_root,
           dmp_bias, gru_w_ih, gru_w_hh, gru_b_ih, gru_b_hh, lin_W, lin_b,
           bn_gamma, bn_beta, sc_W, sc_b):
    src = edge_index[0]
    dst = edge_index[1]
    dst2 = dst.reshape(E // CH, CH)

    def chunk3(idx):
        chunks = idx.reshape(E // CHF, CHF)
        main = chunks[:NW * BASE_CH].reshape(NW, BASE_CH, CHF)
        extras = jnp.zeros((NW, EXTRA, CHF), jnp.int32)
        extras = extras.at[0].set(chunks[NW * BASE_CH:])
        return jnp.concatenate([main, extras], axis=1)

    src3 = chunk3(src)
    dst3 = chunk3(dst)

    mp_b1r = mp_b1.reshape(1, HID)
    mp_b2r = mp_b2.reshape(1, D * D)
    mp_biasr = mp_bias.reshape(1, D)
    dmp_b1r = dmp_b1.reshape(1, HID)
    dmp_b2r = dmp_b2.reshape(1, D * D)
    dmp_biasr = dmp_bias.reshape(1, D)
    wihT = gru_w_ih.T
    whhT = gru_w_hh.T
    bihr = gru_b_ih.reshape(1, 3 * D)
    bhhr = gru_b_hh.reshape(1, 3 * D)
    linbr = lin_b.reshape(1, DOUT)
    gammar = bn_gamma.reshape(1, DOUT)
    betar = bn_beta.reshape(1, DOUT)
    scbr = sc_b.reshape(1, DOUT)

    wam, wbm, wad, wbd = _wprep_call(
        edge_attr, mp_W1, mp_b1r, mp_W2, mp_b2r,
        dmp_W1, dmp_b1r, dmp_W2, dmp_b2r)

    ones_e = jnp.ones((E, D), f32)
    cnt_p = _sc_scatter(ones_e, dst2)[:, :N]
    inv = _inv_call(cnt_p)

    h = x
    cur = x
    for _ in range(3):
        p1 = _sc_pass(cur, src3, dst3, wam, wbm)[:, :N]
        m1 = _combine_call(p1, inv, cur, mp_root, mp_biasr)
        p2 = _sc_pass(m1, src3, dst3, wad, wbd)[:, :N]
        h = _gru_call(p2, inv, m1, h, dmp_root, dmp_biasr, wihT, whhT,
                      bihr, bhhr)
        cur = h

    return _head_call(cur, lin_W, linbr, gammar, betar, sc_W, scbr)


# speedup vs baseline: 4.6514x; 1.5831x over previous
import functools

import jax
import jax.numpy as jnp
from jax import lax
from jax.experimental import pallas as pl
from jax.experimental.pallas import tpu as pltpu
from jax.experimental.pallas import tpu_sc as plsc

N = 10000
E = 160000
D = 16
DE = 16
HID = 64
DOUT = 64

NC = 2
NS = 16
NW = NC * NS
EPT = E // NW
CH = 125
NCHUNK = EPT // CH
N_PAD = 10240
RPT = N_PAD // NS

CHF = 128
BASE_CH = 39
EPT2 = BASE_CH * CHF
EXTRA = 2
MAXCH = BASE_CH + EXTRA

_mesh = plsc.VectorSubcoreMesh(
    core_axis_name="c", subcore_axis_name="s", num_cores=NC, num_subcores=NS)

_sc_params = pltpu.CompilerParams(use_tc_tiling_on_sc=False)

f32 = jnp.float32


@functools.partial(
    pl.kernel,
    out_type=jax.ShapeDtypeStruct((NC, N_PAD, D), f32),
    mesh=_mesh,
    scratch_types=[
        pltpu.VMEM((NCHUNK, CH), jnp.int32),
        pltpu.VMEM((EPT, D), f32),
        pltpu.VMEM_SHARED((N_PAD, D), f32),
        pltpu.SemaphoreType.DMA,
    ],
    compiler_params=_sc_params,
)
def _sc_scatter(msg_hbm, idx2_hbm, out_hbm, idx_v, msg_v, acc_sh, sem):
    cid = lax.axis_index("c")
    sid = lax.axis_index("s")
    wid = sid * NC + cid
    pltpu.sync_copy(idx2_hbm.at[pl.ds(wid * NCHUNK, NCHUNK)], idx_v)

    def zbody(i, carry):
        msg_v[i, :] = jnp.zeros((D,), f32)
        return carry

    lax.fori_loop(0, 128, zbody, 0, unroll=False)

    def zcopy(i, carry):
        pltpu.sync_copy(msg_v.at[pl.ds(0, 128)],
                        acc_sh.at[pl.ds(sid * RPT + i * 128, 128)])
        return carry

    lax.fori_loop(0, RPT // 128, zcopy, 0, unroll=False)
    plsc.subcore_barrier()

    base = wid * EPT
    pltpu.sync_copy(msg_hbm.at[pl.ds(base, EPT)], msg_v)

    def body(j, carry):
        pltpu.sync_copy(msg_v.at[pl.ds(j * CH, CH)],
                        acc_sh.at[idx_v.at[j]], add=True)
        return carry

    lax.fori_loop(0, NCHUNK, body, 0, unroll=False)
    plsc.subcore_barrier()
    pltpu.sync_copy(acc_sh.at[pl.ds(sid * RPT, RPT)],
                    out_hbm.at[cid, pl.ds(sid * RPT, RPT)])


@functools.partial(
    pl.kernel,
    out_type=jax.ShapeDtypeStruct((NC, N_PAD, D), f32),
    mesh=_mesh,
    scratch_types=[
        pltpu.VMEM((MAXCH, CHF), jnp.int32),
        pltpu.VMEM((MAXCH, CHF), jnp.int32),
        pltpu.VMEM((2, CHF, D), f32),
        pltpu.VMEM((2, CHF, 8 * D), f32),
        pltpu.VMEM((2, CHF, 8 * D), f32),
        pltpu.VMEM((CHF, D), f32),
        pltpu.VMEM_SHARED((N_PAD, D), f32),
        pltpu.SemaphoreType.DMA((2,)),
    ],
    compiler_params=_sc_params,
)
def _sc_pass(table_hbm, src3_hbm, dst3_hbm, wa_hbm, wb_hbm, out_hbm,
             sidx_v, didx_v, xs_v, wa_v, wb_v, msg_v, acc_sh, lsem):
    cid = lax.axis_index("c")
    sid = lax.axis_index("s")
    wid = sid * NC + cid
    pltpu.sync_copy(src3_hbm.at[wid], sidx_v)
    pltpu.sync_copy(dst3_hbm.at[wid], didx_v)

    nch = jnp.where(wid == 0, MAXCH, BASE_CH)

    def issue(j, p):
        eoff = jnp.where(j < BASE_CH,
                         wid * EPT2 + j * CHF,
                         NW * EPT2 + (j - BASE_CH) * CHF)
        pltpu.async_copy(wa_hbm.at[pl.ds(eoff, CHF)], wa_v.at[p], lsem.at[p])
        pltpu.async_copy(wb_hbm.at[pl.ds(eoff, CHF)], wb_v.at[p], lsem.at[p])
        pltpu.async_copy(table_hbm.at[sidx_v.at[j]], xs_v.at[p], lsem.at[p])

    issue(0, 0)

    def zbody(i, carry):
        msg_v[i, :] = jnp.zeros((D,), f32)
        return carry

    lax.fori_loop(0, CHF, zbody, 0, unroll=False)

    def zcopy(i, carry):
        pltpu.sync_copy(msg_v, acc_sh.at[pl.ds(sid * RPT + i * CHF, CHF)])
        return carry

    lax.fori_loop(0, RPT // CHF, zcopy, 0, unroll=False)
    plsc.subcore_barrier()

    def body(j, carry):
        p = lax.rem(j, 2)

        @pl.when(j + 1 < nch)
        def _():
            issue(j + 1, 1 - p)

        pltpu.make_async_copy(
            wa_hbm.at[pl.ds(0, CHF)], wa_v.at[p], lsem.at[p]).wait()
        pltpu.make_async_copy(
            wb_hbm.at[pl.ds(0, CHF)], wb_v.at[p], lsem.at[p]).wait()
        pltpu.make_async_copy(
            table_hbm.at[pl.ds(0, CHF)], xs_v.at[p], lsem.at[p]).wait()

        def edge(e, carry2):
            xsrow = xs_v[p, e, :]
            acc = xsrow[0] * wa_v[p, e, 0:D]
            for i in range(1, 8):
                acc = acc + xsrow[i] * wa_v[p, e, i * D:(i + 1) * D]
            for i in range(8):
                acc = acc + xsrow[8 + i] * wb_v[p, e, i * D:(i + 1) * D]
            msg_v[e, :] = acc
            return carry2

        lax.fori_loop(0, CHF, edge, 0, unroll=False)
        pltpu.sync_copy(msg_v, acc_sh.at[didx_v.at[j]], add=True)
        return carry

    lax.fori_loop(0, nch, body, 0, unroll=False)
    plsc.subcore_barrier()
    pltpu.sync_copy(acc_sh.at[pl.ds(sid * RPT, RPT)],
                    out_hbm.at[cid, pl.ds(sid * RPT, RPT)])


BE = 1600
GE = E // BE


def _wprep_body(ea_ref, W1m_ref, b1m_ref, W2m_ref, b2m_ref,
                W1d_ref, b1d_ref, W2d_ref, b2d_ref,
                wam_ref, wbm_ref, wad_ref, wbd_ref):
    ea = ea_ref[...]
    hm = jnp.maximum(
        jnp.dot(ea, W1m_ref[...], preferred_element_type=f32)
        + b1m_ref[...], 0.0)
    wm = jnp.dot(hm, W2m_ref[...], preferred_element_type=f32) + b2m_ref[...]
    wam_ref[...] = wm[:, :8 * D]
    wbm_ref[...] = wm[:, 8 * D:]
    hd = jnp.maximum(
        jnp.dot(ea, W1d_ref[...], preferred_element_type=f32)
        + b1d_ref[...], 0.0)
    wd = jnp.dot(hd, W2d_ref[...], preferred_element_type=f32) + b2d_ref[...]
    wad_ref[...] = wd[:, :8 * D]
    wbd_ref[...] = wd[:, 8 * D:]


_wprep_call = pl.pallas_call(
    _wprep_body,
    grid=(GE,),
    in_specs=[
        pl.BlockSpec((BE, DE), lambda i: (i, 0)),
        pl.BlockSpec((DE, HID), lambda i: (0, 0)),
        pl.BlockSpec((1, HID), lambda i: (0, 0)),
        pl.BlockSpec((HID, D * D), lambda i: (0, 0)),
        pl.BlockSpec((1, D * D), lambda i: (0, 0)),
        pl.BlockSpec((DE, HID), lambda i: (0, 0)),
        pl.BlockSpec((1, HID), lambda i: (0, 0)),
        pl.BlockSpec((HID, D * D), lambda i: (0, 0)),
        pl.BlockSpec((1, D * D), lambda i: (0, 0)),
    ],
    out_specs=[
        pl.BlockSpec((BE, 8 * D), lambda i: (i, 0)),
        pl.BlockSpec((BE, 8 * D), lambda i: (i, 0)),
        pl.BlockSpec((BE, 8 * D), lambda i: (i, 0)),
        pl.BlockSpec((BE, 8 * D), lambda i: (i, 0)),
    ],
    out_shape=[
        jax.ShapeDtypeStruct((E, 8 * D), f32),
        jax.ShapeDtypeStruct((E, 8 * D), f32),
        jax.ShapeDtypeStruct((E, 8 * D), f32),
        jax.ShapeDtypeStruct((E, 8 * D), f32),
    ],
)


def _inv_body(p_ref, out_ref):
    c = p_ref[0] + p_ref[1]
    out_ref[...] = 1.0 / jnp.maximum(c, 1.0)


_inv_call = pl.pallas_call(
    _inv_body,
    out_shape=jax.ShapeDtypeStruct((N, D), f32),
)


def _combine_body(p_ref, inv_ref, cur_ref, root_ref, bias_ref, out_ref):
    aggr = (p_ref[0] + p_ref[1]) * inv_ref[...]
    out_ref[...] = aggr + jnp.dot(
        cur_ref[...], root_ref[...], preferred_element_type=f32) + bias_ref[...]


_combine_call = pl.pallas_call(
    _combine_body,
    out_shape=jax.ShapeDtypeStruct((N, D), f32),
)


def _gru_body(p_ref, inv_ref, m1_ref, h_ref, root_ref, bias_ref, wihT_ref,
              whhT_ref, bih_ref, bhh_ref, out_ref):
    m1 = m1_ref[...]
    h = h_ref[...]
    m2 = ((p_ref[0] + p_ref[1]) * inv_ref[...]
          + jnp.dot(m1, root_ref[...], preferred_element_type=f32)
          + bias_ref[...])
    gi = jnp.dot(m2, wihT_ref[...], preferred_element_type=f32) + bih_ref[...]
    gh = jnp.dot(h, whhT_ref[...], preferred_element_type=f32) + bhh_ref[...]
    r = jax.nn.sigmoid(gi[:, :D] + gh[:, :D])
    z = jax.nn.sigmoid(gi[:, D:2 * D] + gh[:, D:2 * D])
    n_ = jnp.tanh(gi[:, 2 * D:] + r * gh[:, 2 * D:])
    out_ref[...] = (1.0 - z) * n_ + z * h


_gru_call = pl.pallas_call(
    _gru_body,
    out_shape=jax.ShapeDtypeStruct((N, D), f32),
)


def _head_body(cur_ref, linW_ref, linb_ref, g_ref, b_ref, scW_ref, scb_ref,
               out_ref):
    cur = cur_ref[...]
    y = jnp.dot(cur, linW_ref[...], preferred_element_type=f32) + linb_ref[...]
    mean = jnp.mean(y, axis=0, keepdims=True)
    var = jnp.mean((y - mean) ** 2, axis=0, keepdims=True)
    yn = (y - mean) / jnp.sqrt(var + 1e-5) * g_ref[...] + b_ref[...]
    out_ref[...] = jnp.maximum(yn, 0.0) + jnp.dot(
        cur, scW_ref[...], preferred_element_type=f32) + scb_ref[...]


_head_call = pl.pallas_call(
    _head_body,
    out_shape=jax.ShapeDtypeStruct((N, DOUT), f32),
)


def kernel(x, edge_index, edge_attr, batch, mp_W1, mp_b1, mp_W2, mp_b2,
           mp_root, mp_bias, dmp_W1, dmp_b1, dmp_W2, dmp_b2, dmp_root,
           dmp_bias, gru_w_ih, gru_w_hh, gru_b_ih, gru_b_hh, lin_W, lin_b,
           bn_gamma, bn_beta, sc_W, sc_b):
    src = edge_index[0]
    dst = edge_index[1]
    dst2 = dst.reshape(E // CH, CH)

    def chunk3(idx):
        chunks = idx.reshape(E // CHF, CHF)
        main = chunks[:NW * BASE_CH].reshape(NW, BASE_CH, CHF)
        extras = jnp.zeros((NW, EXTRA, CHF), jnp.int32)
        extras = extras.at[0].set(chunks[NW * BASE_CH:])
        return jnp.concatenate([main, extras], axis=1)

    src3 = chunk3(src)
    dst3 = chunk3(dst)

    mp_b1r = mp_b1.reshape(1, HID)
    mp_b2r = mp_b2.reshape(1, D * D)
    mp_biasr = mp_bias.reshape(1, D)
    dmp_b1r = dmp_b1.reshape(1, HID)
    dmp_b2r = dmp_b2.reshape(1, D * D)
    dmp_biasr = dmp_bias.reshape(1, D)
    wihT = gru_w_ih.T
    whhT = gru_w_hh.T
    bihr = gru_b_ih.reshape(1, 3 * D)
    bhhr = gru_b_hh.reshape(1, 3 * D)
    linbr = lin_b.reshape(1, DOUT)
    gammar = bn_gamma.reshape(1, DOUT)
    betar = bn_beta.reshape(1, DOUT)
    scbr = sc_b.reshape(1, DOUT)

    wam, wbm, wad, wbd = _wprep_call(
        edge_attr, mp_W1, mp_b1r, mp_W2, mp_b2r,
        dmp_W1, dmp_b1r, dmp_W2, dmp_b2r)

    ones_e = jnp.ones((E, D), f32)
    cnt_p = _sc_scatter(ones_e, dst2)[:, :N]
    inv = _inv_call(cnt_p)

    h = x
    cur = x
    for _ in range(3):
        p1 = _sc_pass(cur, src3, dst3, wam, wbm)[:, :N]
        m1 = _combine_call(p1, inv, cur, mp_root, mp_biasr)
        p2 = _sc_pass(m1, src3, dst3, wad, wbd)[:, :N]
        h = _gru_call(p2, inv, m1, h, dmp_root, dmp_biasr, wihT, whhT,
                      bihr, bhhr)
        cur = h

    return _head_call(cur, lin_W, linbr, gammar, betar, sc_W, scbr)


# speedup vs baseline: 4.6751x; 1.0051x over previous
import functools

import jax
import jax.numpy as jnp
from jax import lax
from jax.experimental import pallas as pl
from jax.experimental.pallas import tpu as pltpu
from jax.experimental.pallas import tpu_sc as plsc

N = 10000
E = 160000
D = 16
DE = 16
HID = 64
DOUT = 64

NC = 2
NS = 16
NW = NC * NS
EPT = E // NW
CH = 125
NCHUNK = EPT // CH
N_PAD = 10240
RPT = N_PAD // NS

CHF = 128
BASE_CH = 39
EPT2 = BASE_CH * CHF
EXTRA = 2
MAXCH = BASE_CH + EXTRA

_mesh = plsc.VectorSubcoreMesh(
    core_axis_name="c", subcore_axis_name="s", num_cores=NC, num_subcores=NS)

_sc_params = pltpu.CompilerParams(use_tc_tiling_on_sc=False)

f32 = jnp.float32


@functools.partial(
    pl.kernel,
    out_type=jax.ShapeDtypeStruct((NC, N_PAD, D), f32),
    mesh=_mesh,
    scratch_types=[
        pltpu.VMEM((MAXCH, CHF), jnp.int32),
        pltpu.VMEM((CHF, D), f32),
        pltpu.VMEM_SHARED((N_PAD, D), f32),
        pltpu.SemaphoreType.DMA,
    ],
    compiler_params=_sc_params,
)
def _sc_count(dst3_hbm, out_hbm, didx_v, ones_v, acc_sh, sem):
    cid = lax.axis_index("c")
    sid = lax.axis_index("s")
    wid = sid * NC + cid
    pltpu.sync_copy(dst3_hbm.at[wid], didx_v)

    def zbody(i, carry):
        ones_v[i, :] = jnp.zeros((D,), f32)
        return carry

    lax.fori_loop(0, CHF, zbody, 0, unroll=False)

    def zcopy(i, carry):
        pltpu.sync_copy(ones_v, acc_sh.at[pl.ds(sid * RPT + i * CHF, CHF)])
        return carry

    lax.fori_loop(0, RPT // CHF, zcopy, 0, unroll=False)

    def obody(i, carry):
        ones_v[i, :] = jnp.ones((D,), f32)
        return carry

    lax.fori_loop(0, CHF, obody, 0, unroll=False)
    plsc.subcore_barrier()

    nch = jnp.where(wid == 0, MAXCH, BASE_CH)

    def body(j, carry):
        pltpu.sync_copy(ones_v, acc_sh.at[didx_v.at[j]], add=True)
        return carry

    lax.fori_loop(0, nch, body, 0, unroll=False)
    plsc.subcore_barrier()
    pltpu.sync_copy(acc_sh.at[pl.ds(sid * RPT, RPT)],
                    out_hbm.at[cid, pl.ds(sid * RPT, RPT)])


@functools.partial(
    pl.kernel,
    out_type=jax.ShapeDtypeStruct((NC, N_PAD, D), f32),
    mesh=_mesh,
    scratch_types=[
        pltpu.VMEM((MAXCH, CHF), jnp.int32),
        pltpu.VMEM((MAXCH, CHF), jnp.int32),
        pltpu.VMEM((2, CHF, D), f32),
        pltpu.VMEM((2, CHF, 8 * D), f32),
        pltpu.VMEM((2, CHF, 8 * D), f32),
        pltpu.VMEM((CHF, D), f32),
        pltpu.VMEM_SHARED((N_PAD, D), f32),
        pltpu.SemaphoreType.DMA((2,)),
    ],
    compiler_params=_sc_params,
)
def _sc_pass(table_hbm, src3_hbm, dst3_hbm, wa_hbm, wb_hbm, out_hbm,
             sidx_v, didx_v, xs_v, wa_v, wb_v, msg_v, acc_sh, lsem):
    cid = lax.axis_index("c")
    sid = lax.axis_index("s")
    wid = sid * NC + cid
    pltpu.sync_copy(src3_hbm.at[wid], sidx_v)
    pltpu.sync_copy(dst3_hbm.at[wid], didx_v)

    nch = jnp.where(wid == 0, MAXCH, BASE_CH)

    def issue(j, p):
        eoff = jnp.where(j < BASE_CH,
                         wid * EPT2 + j * CHF,
                         NW * EPT2 + (j - BASE_CH) * CHF)
        pltpu.async_copy(wa_hbm.at[pl.ds(eoff, CHF)], wa_v.at[p], lsem.at[p])
        pltpu.async_copy(wb_hbm.at[pl.ds(eoff, CHF)], wb_v.at[p], lsem.at[p])
        pltpu.async_copy(table_hbm.at[sidx_v.at[j]], xs_v.at[p], lsem.at[p])

    issue(0, 0)

    def zbody(i, carry):
        msg_v[i, :] = jnp.zeros((D,), f32)
        return carry

    lax.fori_loop(0, CHF, zbody, 0, unroll=False)

    def zcopy(i, carry):
        pltpu.sync_copy(msg_v, acc_sh.at[pl.ds(sid * RPT + i * CHF, CHF)])
        return carry

    lax.fori_loop(0, RPT // CHF, zcopy, 0, unroll=False)
    plsc.subcore_barrier()

    def body(j, carry):
        p = lax.rem(j, 2)

        @pl.when(j + 1 < nch)
        def _():
            issue(j + 1, 1 - p)

        pltpu.make_async_copy(
            wa_hbm.at[pl.ds(0, CHF)], wa_v.at[p], lsem.at[p]).wait()
        pltpu.make_async_copy(
            wb_hbm.at[pl.ds(0, CHF)], wb_v.at[p], lsem.at[p]).wait()
        pltpu.make_async_copy(
            table_hbm.at[pl.ds(0, CHF)], xs_v.at[p], lsem.at[p]).wait()

        def edge(e, carry2):
            xsrow = xs_v[p, e, :]
            acc = xsrow[0] * wa_v[p, e, 0:D]
            for i in range(1, 8):
                acc = acc + xsrow[i] * wa_v[p, e, i * D:(i + 1) * D]
            for i in range(8):
                acc = acc + xsrow[8 + i] * wb_v[p, e, i * D:(i + 1) * D]
            msg_v[e, :] = acc
            return carry2

        lax.fori_loop(0, CHF, edge, 0, unroll=False)
        pltpu.sync_copy(msg_v, acc_sh.at[didx_v.at[j]], add=True)
        return carry

    lax.fori_loop(0, nch, body, 0, unroll=False)
    plsc.subcore_barrier()
    pltpu.sync_copy(acc_sh.at[pl.ds(sid * RPT, RPT)],
                    out_hbm.at[cid, pl.ds(sid * RPT, RPT)])


BE = 1600
GE = E // BE


def _wprep_body(ea_ref, W1m_ref, b1m_ref, W2m_ref, b2m_ref,
                W1d_ref, b1d_ref, W2d_ref, b2d_ref,
                wam_ref, wbm_ref, wad_ref, wbd_ref):
    ea = ea_ref[...]
    hm = jnp.maximum(
        jnp.dot(ea, W1m_ref[...], preferred_element_type=f32)
        + b1m_ref[...], 0.0)
    wm = jnp.dot(hm, W2m_ref[...], preferred_element_type=f32) + b2m_ref[...]
    wam_ref[...] = wm[:, :8 * D]
    wbm_ref[...] = wm[:, 8 * D:]
    hd = jnp.maximum(
        jnp.dot(ea, W1d_ref[...], preferred_element_type=f32)
        + b1d_ref[...], 0.0)
    wd = jnp.dot(hd, W2d_ref[...], preferred_element_type=f32) + b2d_ref[...]
    wad_ref[...] = wd[:, :8 * D]
    wbd_ref[...] = wd[:, 8 * D:]


_wprep_call = pl.pallas_call(
    _wprep_body,
    grid=(GE,),
    in_specs=[
        pl.BlockSpec((BE, DE), lambda i: (i, 0)),
        pl.BlockSpec((DE, HID), lambda i: (0, 0)),
        pl.BlockSpec((1, HID), lambda i: (0, 0)),
        pl.BlockSpec((HID, D * D), lambda i: (0, 0)),
        pl.BlockSpec((1, D * D), lambda i: (0, 0)),
        pl.BlockSpec((DE, HID), lambda i: (0, 0)),
        pl.BlockSpec((1, HID), lambda i: (0, 0)),
        pl.BlockSpec((HID, D * D), lambda i: (0, 0)),
        pl.BlockSpec((1, D * D), lambda i: (0, 0)),
    ],
    out_specs=[
        pl.BlockSpec((BE, 8 * D), lambda i: (i, 0)),
        pl.BlockSpec((BE, 8 * D), lambda i: (i, 0)),
        pl.BlockSpec((BE, 8 * D), lambda i: (i, 0)),
        pl.BlockSpec((BE, 8 * D), lambda i: (i, 0)),
    ],
    out_shape=[
        jax.ShapeDtypeStruct((E, 8 * D), f32),
        jax.ShapeDtypeStruct((E, 8 * D), f32),
        jax.ShapeDtypeStruct((E, 8 * D), f32),
        jax.ShapeDtypeStruct((E, 8 * D), f32),
    ],
)


def _inv_body(p_ref, out_ref):
    c = p_ref[0] + p_ref[1]
    out_ref[...] = 1.0 / jnp.maximum(c, 1.0)


_inv_call = pl.pallas_call(
    _inv_body,
    out_shape=jax.ShapeDtypeStruct((N, D), f32),
)


def _combine_body(p_ref, inv_ref, cur_ref, root_ref, bias_ref, out_ref):
    aggr = (p_ref[0] + p_ref[1]) * inv_ref[...]
    out_ref[...] = aggr + jnp.dot(
        cur_ref[...], root_ref[...], preferred_element_type=f32) + bias_ref[...]


_combine_call = pl.pallas_call(
    _combine_body,
    out_shape=jax.ShapeDtypeStruct((N, D), f32),
)


def _gru_body(p_ref, inv_ref, m1_ref, h_ref, root_ref, bias_ref, wihT_ref,
              whhT_ref, bih_ref, bhh_ref, out_ref):
    m1 = m1_ref[...]
    h = h_ref[...]
    m2 = ((p_ref[0] + p_ref[1]) * inv_ref[...]
          + jnp.dot(m1, root_ref[...], preferred_element_type=f32)
          + bias_ref[...])
    gi = jnp.dot(m2, wihT_ref[...], preferred_element_type=f32) + bih_ref[...]
    gh = jnp.dot(h, whhT_ref[...], preferred_element_type=f32) + bhh_ref[...]
    r = jax.nn.sigmoid(gi[:, :D] + gh[:, :D])
    z = jax.nn.sigmoid(gi[:, D:2 * D] + gh[:, D:2 * D])
    n_ = jnp.tanh(gi[:, 2 * D:] + r * gh[:, 2 * D:])
    out_ref[...] = (1.0 - z) * n_ + z * h


_gru_call = pl.pallas_call(
    _gru_body,
    out_shape=jax.ShapeDtypeStruct((N, D), f32),
)


def _head_body(cur_ref, linW_ref, linb_ref, g_ref, b_ref, scW_ref, scb_ref,
               out_ref):
    cur = cur_ref[...]
    y = jnp.dot(cur, linW_ref[...], preferred_element_type=f32) + linb_ref[...]
    mean = jnp.mean(y, axis=0, keepdims=True)
    var = jnp.mean((y - mean) ** 2, axis=0, keepdims=True)
    yn = (y - mean) / jnp.sqrt(var + 1e-5) * g_ref[...] + b_ref[...]
    out_ref[...] = jnp.maximum(yn, 0.0) + jnp.dot(
        cur, scW_ref[...], preferred_element_type=f32) + scb_ref[...]


_head_call = pl.pallas_call(
    _head_body,
    out_shape=jax.ShapeDtypeStruct((N, DOUT), f32),
)


def kernel(x, edge_index, edge_attr, batch, mp_W1, mp_b1, mp_W2, mp_b2,
           mp_root, mp_bias, dmp_W1, dmp_b1, dmp_W2, dmp_b2, dmp_root,
           dmp_bias, gru_w_ih, gru_w_hh, gru_b_ih, gru_b_hh, lin_W, lin_b,
           bn_gamma, bn_beta, sc_W, sc_b):
    src = edge_index[0]
    dst = edge_index[1]

    def chunk3(idx):
        chunks = idx.reshape(E // CHF, CHF)
        main = chunks[:NW * BASE_CH].reshape(NW, BASE_CH, CHF)
        extras = jnp.zeros((NW, EXTRA, CHF), jnp.int32)
        extras = extras.at[0].set(chunks[NW * BASE_CH:])
        return jnp.concatenate([main, extras], axis=1)

    src3 = chunk3(src)
    dst3 = chunk3(dst)

    mp_b1r = mp_b1.reshape(1, HID)
    mp_b2r = mp_b2.reshape(1, D * D)
    mp_biasr = mp_bias.reshape(1, D)
    dmp_b1r = dmp_b1.reshape(1, HID)
    dmp_b2r = dmp_b2.reshape(1, D * D)
    dmp_biasr = dmp_bias.reshape(1, D)
    wihT = gru_w_ih.T
    whhT = gru_w_hh.T
    bihr = gru_b_ih.reshape(1, 3 * D)
    bhhr = gru_b_hh.reshape(1, 3 * D)
    linbr = lin_b.reshape(1, DOUT)
    gammar = bn_gamma.reshape(1, DOUT)
    betar = bn_beta.reshape(1, DOUT)
    scbr = sc_b.reshape(1, DOUT)

    wam, wbm, wad, wbd = _wprep_call(
        edge_attr, mp_W1, mp_b1r, mp_W2, mp_b2r,
        dmp_W1, dmp_b1r, dmp_W2, dmp_b2r)

    cnt_p = _sc_count(dst3)[:, :N]
    inv = _inv_call(cnt_p)

    h = x
    cur = x
    for _ in range(3):
        p1 = _sc_pass(cur, src3, dst3, wam, wbm)[:, :N]
        m1 = _combine_call(p1, inv, cur, mp_root, mp_biasr)
        p2 = _sc_pass(m1, src3, dst3, wad, wbd)[:, :N]
        h = _gru_call(p2, inv, m1, h, dmp_root, dmp_biasr, wihT, whhT,
                      bihr, bhhr)
        cur = h

    return _head_call(cur, lin_W, linbr, gammar, betar, sc_W, scbr)


# speedup vs baseline: 4.7925x; 1.0251x over previous
import functools

import jax
import jax.numpy as jnp
from jax import lax
from jax.experimental import pallas as pl
from jax.experimental.pallas import tpu as pltpu
from jax.experimental.pallas import tpu_sc as plsc

N = 10000
E = 160000
D = 16
DE = 16
HID = 64
DOUT = 64

NC = 2
NS = 16
NW = NC * NS
EPT = E // NW
CH = 125
NCHUNK = EPT // CH
N_PAD = 10240
RPT = N_PAD // NS

CHF = 128
BASE_CH = 39
EPT2 = BASE_CH * CHF
EXTRA = 2
MAXCH = BASE_CH + EXTRA

_mesh = plsc.VectorSubcoreMesh(
    core_axis_name="c", subcore_axis_name="s", num_cores=NC, num_subcores=NS)

_sc_params = pltpu.CompilerParams(use_tc_tiling_on_sc=False)

f32 = jnp.float32


@functools.partial(
    pl.kernel,
    out_type=jax.ShapeDtypeStruct((NC, N_PAD, D), f32),
    mesh=_mesh,
    scratch_types=[
        pltpu.VMEM((MAXCH, CHF), jnp.int32),
        pltpu.VMEM((CHF, D), f32),
        pltpu.VMEM_SHARED((N_PAD, D), f32),
        pltpu.SemaphoreType.DMA,
    ],
    compiler_params=_sc_params,
)
def _sc_count(dst3_hbm, out_hbm, didx_v, ones_v, acc_sh, sem):
    cid = lax.axis_index("c")
    sid = lax.axis_index("s")
    wid = sid * NC + cid
    pltpu.sync_copy(dst3_hbm.at[wid], didx_v)

    def zbody(i, carry):
        ones_v[i, :] = jnp.zeros((D,), f32)
        return carry

    lax.fori_loop(0, CHF, zbody, 0, unroll=False)

    def zcopy(i, carry):
        pltpu.sync_copy(ones_v, acc_sh.at[pl.ds(sid * RPT + i * CHF, CHF)])
        return carry

    lax.fori_loop(0, RPT // CHF, zcopy, 0, unroll=False)

    def obody(i, carry):
        ones_v[i, :] = jnp.ones((D,), f32)
        return carry

    lax.fori_loop(0, CHF, obody, 0, unroll=False)
    plsc.subcore_barrier()

    nch = jnp.where(wid == 0, MAXCH, BASE_CH)

    def body(j, carry):
        pltpu.sync_copy(ones_v, acc_sh.at[didx_v.at[j]], add=True)
        return carry

    lax.fori_loop(0, nch, body, 0, unroll=False)
    plsc.subcore_barrier()
    pltpu.sync_copy(acc_sh.at[pl.ds(sid * RPT, RPT)],
                    out_hbm.at[cid, pl.ds(sid * RPT, RPT)])


@functools.partial(
    pl.kernel,
    out_type=jax.ShapeDtypeStruct((NC, N_PAD, D), f32),
    mesh=_mesh,
    scratch_types=[
        pltpu.VMEM((MAXCH, CHF), jnp.int32),
        pltpu.VMEM((MAXCH, CHF), jnp.int32),
        pltpu.VMEM((2, CHF, D), f32),
        pltpu.VMEM((2, CHF, 8 * D), f32),
        pltpu.VMEM((2, CHF, 8 * D), f32),
        pltpu.VMEM((2 * CHF, D), f32),
        pltpu.VMEM_SHARED((N_PAD, D), f32),
        pltpu.SemaphoreType.DMA((2,)),
        pltpu.SemaphoreType.DMA((2,)),
    ],
    compiler_params=_sc_params,
)
def _sc_pass(table_hbm, src3_hbm, dst3_hbm, wa_hbm, wb_hbm, out_hbm,
             sidx_v, didx_v, xs_v, wa_v, wb_v, msg_v, acc_sh, lsem, ssem):
    cid = lax.axis_index("c")
    sid = lax.axis_index("s")
    wid = sid * NC + cid
    pltpu.sync_copy(src3_hbm.at[wid], sidx_v)
    pltpu.sync_copy(dst3_hbm.at[wid], didx_v)

    nch = jnp.where(wid == 0, MAXCH, BASE_CH)

    def issue(j, p):
        eoff = jnp.where(j < BASE_CH,
                         wid * EPT2 + j * CHF,
                         NW * EPT2 + (j - BASE_CH) * CHF)
        pltpu.async_copy(wa_hbm.at[pl.ds(eoff, CHF)], wa_v.at[p], lsem.at[p])
        pltpu.async_copy(wb_hbm.at[pl.ds(eoff, CHF)], wb_v.at[p], lsem.at[p])
        pltpu.async_copy(table_hbm.at[sidx_v.at[j]], xs_v.at[p], lsem.at[p])

    issue(0, 0)

    def zbody(i, carry):
        msg_v[i, :] = jnp.zeros((D,), f32)
        return carry

    lax.fori_loop(0, CHF, zbody, 0, unroll=False)

    def zcopy(i, carry):
        pltpu.sync_copy(msg_v.at[pl.ds(0, CHF)],
                        acc_sh.at[pl.ds(sid * RPT + i * CHF, CHF)])
        return carry

    lax.fori_loop(0, RPT // CHF, zcopy, 0, unroll=False)
    plsc.subcore_barrier()

    def body(j, carry):
        p = lax.rem(j, 2)

        @pl.when(j + 1 < nch)
        def _():
            issue(j + 1, 1 - p)

        pltpu.make_async_copy(
            wa_hbm.at[pl.ds(0, CHF)], wa_v.at[p], lsem.at[p]).wait()
        pltpu.make_async_copy(
            wb_hbm.at[pl.ds(0, CHF)], wb_v.at[p], lsem.at[p]).wait()
        pltpu.make_async_copy(
            table_hbm.at[pl.ds(0, CHF)], xs_v.at[p], lsem.at[p]).wait()

        @pl.when(j >= 2)
        def _():
            pltpu.make_async_copy(
                msg_v.at[pl.ds(p * CHF, CHF)], acc_sh.at[didx_v.at[j - 2]],
                ssem.at[p]).wait()

        def edge(e, carry2):
            xsrow = xs_v[p, e, :]
            acc = xsrow[0] * wa_v[p, e, 0:D]
            for i in range(1, 8):
                acc = acc + xsrow[i] * wa_v[p, e, i * D:(i + 1) * D]
            for i in range(8):
                acc = acc + xsrow[8 + i] * wb_v[p, e, i * D:(i + 1) * D]
            msg_v[p * CHF + e, :] = acc
            return carry2

        lax.fori_loop(0, CHF, edge, 0, unroll=False)
        pltpu.async_copy(msg_v.at[pl.ds(p * CHF, CHF)],
                         acc_sh.at[didx_v.at[j]], ssem.at[p], add=True)
        return carry

    lax.fori_loop(0, nch, body, 0, unroll=False)

    def drain(p, carry):
        pltpu.make_async_copy(
            msg_v.at[pl.ds(p * CHF, CHF)], acc_sh.at[didx_v.at[0]],
            ssem.at[p]).wait()
        return carry

    lax.fori_loop(0, 2, drain, 0, unroll=False)
    plsc.subcore_barrier()
    pltpu.sync_copy(acc_sh.at[pl.ds(sid * RPT, RPT)],
                    out_hbm.at[cid, pl.ds(sid * RPT, RPT)])


BE = 1600
GE = E // BE


def _wprep_body(ea_ref, W1m_ref, b1m_ref, W2m_ref, b2m_ref,
                W1d_ref, b1d_ref, W2d_ref, b2d_ref,
                wam_ref, wbm_ref, wad_ref, wbd_ref):
    ea = ea_ref[...]
    hm = jnp.maximum(
        jnp.dot(ea, W1m_ref[...], preferred_element_type=f32)
        + b1m_ref[...], 0.0)
    wm = jnp.dot(hm, W2m_ref[...], preferred_element_type=f32) + b2m_ref[...]
    wam_ref[...] = wm[:, :8 * D]
    wbm_ref[...] = wm[:, 8 * D:]
    hd = jnp.maximum(
        jnp.dot(ea, W1d_ref[...], preferred_element_type=f32)
        + b1d_ref[...], 0.0)
    wd = jnp.dot(hd, W2d_ref[...], preferred_element_type=f32) + b2d_ref[...]
    wad_ref[...] = wd[:, :8 * D]
    wbd_ref[...] = wd[:, 8 * D:]


_wprep_call = pl.pallas_call(
    _wprep_body,
    grid=(GE,),
    in_specs=[
        pl.BlockSpec((BE, DE), lambda i: (i, 0)),
        pl.BlockSpec((DE, HID), lambda i: (0, 0)),
        pl.BlockSpec((1, HID), lambda i: (0, 0)),
        pl.BlockSpec((HID, D * D), lambda i: (0, 0)),
        pl.BlockSpec((1, D * D), lambda i: (0, 0)),
        pl.BlockSpec((DE, HID), lambda i: (0, 0)),
        pl.BlockSpec((1, HID), lambda i: (0, 0)),
        pl.BlockSpec((HID, D * D), lambda i: (0, 0)),
        pl.BlockSpec((1, D * D), lambda i: (0, 0)),
    ],
    out_specs=[
        pl.BlockSpec((BE, 8 * D), lambda i: (i, 0)),
        pl.BlockSpec((BE, 8 * D), lambda i: (i, 0)),
        pl.BlockSpec((BE, 8 * D), lambda i: (i, 0)),
        pl.BlockSpec((BE, 8 * D), lambda i: (i, 0)),
    ],
    out_shape=[
        jax.ShapeDtypeStruct((E, 8 * D), f32),
        jax.ShapeDtypeStruct((E, 8 * D), f32),
        jax.ShapeDtypeStruct((E, 8 * D), f32),
        jax.ShapeDtypeStruct((E, 8 * D), f32),
    ],
)


def _inv_body(p_ref, out_ref):
    c = p_ref[0] + p_ref[1]
    out_ref[...] = 1.0 / jnp.maximum(c, 1.0)


_inv_call = pl.pallas_call(
    _inv_body,
    out_shape=jax.ShapeDtypeStruct((N, D), f32),
)


def _combine_body(p_ref, inv_ref, cur_ref, root_ref, bias_ref, out_ref):
    aggr = (p_ref[0] + p_ref[1]) * inv_ref[...]
    out_ref[...] = aggr + jnp.dot(
        cur_ref[...], root_ref[...], preferred_element_type=f32) + bias_ref[...]


_combine_call = pl.pallas_call(
    _combine_body,
    out_shape=jax.ShapeDtypeStruct((N, D), f32),
)


def _gru_body(p_ref, inv_ref, m1_ref, h_ref, root_ref, bias_ref, wihT_ref,
              whhT_ref, bih_ref, bhh_ref, out_ref):
    m1 = m1_ref[...]
    h = h_ref[...]
    m2 = ((p_ref[0] + p_ref[1]) * inv_ref[...]
          + jnp.dot(m1, root_ref[...], preferred_element_type=f32)
          + bias_ref[...])
    gi = jnp.dot(m2, wihT_ref[...], preferred_element_type=f32) + bih_ref[...]
    gh = jnp.dot(h, whhT_ref[...], preferred_element_type=f32) + bhh_ref[...]
    r = jax.nn.sigmoid(gi[:, :D] + gh[:, :D])
    z = jax.nn.sigmoid(gi[:, D:2 * D] + gh[:, D:2 * D])
    n_ = jnp.tanh(gi[:, 2 * D:] + r * gh[:, 2 * D:])
    out_ref[...] = (1.0 - z) * n_ + z * h


_gru_call = pl.pallas_call(
    _gru_body,
    out_shape=jax.ShapeDtypeStruct((N, D), f32),
)


def _head_body(cur_ref, linW_ref, linb_ref, g_ref, b_ref, scW_ref, scb_ref,
               out_ref):
    cur = cur_ref[...]
    y = jnp.dot(cur, linW_ref[...], preferred_element_type=f32) + linb_ref[...]
    mean = jnp.mean(y, axis=0, keepdims=True)
    var = jnp.mean((y - mean) ** 2, axis=0, keepdims=True)
    yn = (y - mean) / jnp.sqrt(var + 1e-5) * g_ref[...] + b_ref[...]
    out_ref[...] = jnp.maximum(yn, 0.0) + jnp.dot(
        cur, scW_ref[...], preferred_element_type=f32) + scb_ref[...]


_head_call = pl.pallas_call(
    _head_body,
    out_shape=jax.ShapeDtypeStruct((N, DOUT), f32),
)


def kernel(x, edge_index, edge_attr, batch, mp_W1, mp_b1, mp_W2, mp_b2,
           mp_root, mp_bias, dmp_W1, dmp_b1, dmp_W2, dmp_b2, dmp_root,
           dmp_bias, gru_w_ih, gru_w_hh, gru_b_ih, gru_b_hh, lin_W, lin_b,
           bn_gamma, bn_beta, sc_W, sc_b):
    src = edge_index[0]
    dst = edge_index[1]

    def chunk3(idx):
        chunks = idx.reshape(E // CHF, CHF)
        main = chunks[:NW * BASE_CH].reshape(NW, BASE_CH, CHF)
        extras = jnp.zeros((NW, EXTRA, CHF), jnp.int32)
        extras = extras.at[0].set(chunks[NW * BASE_CH:])
        return jnp.concatenate([main, extras], axis=1)

    src3 = chunk3(src)
    dst3 = chunk3(dst)

    mp_b1r = mp_b1.reshape(1, HID)
    mp_b2r = mp_b2.reshape(1, D * D)
    mp_biasr = mp_bias.reshape(1, D)
    dmp_b1r = dmp_b1.reshape(1, HID)
    dmp_b2r = dmp_b2.reshape(1, D * D)
    dmp_biasr = dmp_bias.reshape(1, D)
    wihT = gru_w_ih.T
    whhT = gru_w_hh.T
    bihr = gru_b_ih.reshape(1, 3 * D)
    bhhr = gru_b_hh.reshape(1, 3 * D)
    linbr = lin_b.reshape(1, DOUT)
    gammar = bn_gamma.reshape(1, DOUT)
    betar = bn_beta.reshape(1, DOUT)
    scbr = sc_b.reshape(1, DOUT)

    wam, wbm, wad, wbd = _wprep_call(
        edge_attr, mp_W1, mp_b1r, mp_W2, mp_b2r,
        dmp_W1, dmp_b1r, dmp_W2, dmp_b2r)

    cnt_p = _sc_count(dst3)[:, :N]
    inv = _inv_call(cnt_p)

    h = x
    cur = x
    for _ in range(3):
        p1 = _sc_pass(cur, src3, dst3, wam, wbm)[:, :N]
        m1 = _combine_call(p1, inv, cur, mp_root, mp_biasr)
        p2 = _sc_pass(m1, src3, dst3, wad, wbd)[:, :N]
        h = _gru_call(p2, inv, m1, h, dmp_root, dmp_biasr, wihT, whhT,
                      bihr, bhhr)
        cur = h

    return _head_call(cur, lin_W, linbr, gammar, betar, sc_W, scbr)


# speedup vs baseline: 4.9511x; 1.0331x over previous
import functools

import jax
import jax.numpy as jnp
from jax import lax
from jax.experimental import pallas as pl
from jax.experimental.pallas import tpu as pltpu
from jax.experimental.pallas import tpu_sc as plsc

N = 10000
E = 160000
D = 16
DE = 16
HID = 64
DOUT = 64

NC = 2
NS = 16
NW = NC * NS
EPT = E // NW
CH = 125
NCHUNK = EPT // CH
N_PAD = 10240
RPT = N_PAD // NS

CHF = 128
BASE_CH = 39
EPT2 = BASE_CH * CHF
EXTRA = 2
MAXCH = BASE_CH + EXTRA

_mesh = plsc.VectorSubcoreMesh(
    core_axis_name="c", subcore_axis_name="s", num_cores=NC, num_subcores=NS)

_sc_params = pltpu.CompilerParams(use_tc_tiling_on_sc=False)

f32 = jnp.float32


@functools.partial(
    pl.kernel,
    out_type=jax.ShapeDtypeStruct((NC, N_PAD, D), f32),
    mesh=_mesh,
    scratch_types=[
        pltpu.VMEM((MAXCH, CHF), jnp.int32),
        pltpu.VMEM((CHF, D), f32),
        pltpu.VMEM_SHARED((N_PAD, D), f32),
        pltpu.SemaphoreType.DMA,
    ],
    compiler_params=_sc_params,
)
def _sc_count(dst3_hbm, out_hbm, didx_v, ones_v, acc_sh, sem):
    cid = lax.axis_index("c")
    sid = lax.axis_index("s")
    wid = sid * NC + cid
    pltpu.sync_copy(dst3_hbm.at[wid], didx_v)

    def zbody(i, carry):
        ones_v[i, :] = jnp.zeros((D,), f32)
        return carry

    lax.fori_loop(0, CHF, zbody, 0, unroll=False)

    def zcopy(i, carry):
        pltpu.sync_copy(ones_v, acc_sh.at[pl.ds(sid * RPT + i * CHF, CHF)])
        return carry

    lax.fori_loop(0, RPT // CHF, zcopy, 0, unroll=False)

    def obody(i, carry):
        ones_v[i, :] = jnp.ones((D,), f32)
        return carry

    lax.fori_loop(0, CHF, obody, 0, unroll=False)
    plsc.subcore_barrier()

    nch = jnp.where(wid == 0, MAXCH, BASE_CH)

    def body(j, carry):
        pltpu.sync_copy(ones_v, acc_sh.at[didx_v.at[j]], add=True)
        return carry

    lax.fori_loop(0, nch, body, 0, unroll=False)
    plsc.subcore_barrier()
    pltpu.sync_copy(acc_sh.at[pl.ds(sid * RPT, RPT)],
                    out_hbm.at[cid, pl.ds(sid * RPT, RPT)])


@functools.partial(
    pl.kernel,
    out_type=jax.ShapeDtypeStruct((NC, N_PAD, D), f32),
    mesh=_mesh,
    scratch_types=[
        pltpu.VMEM((MAXCH, CHF), jnp.int32),
        pltpu.VMEM((MAXCH, CHF), jnp.int32),
        pltpu.VMEM((2, CHF, D), f32),
        pltpu.VMEM((2, CHF, 8 * D), f32),
        pltpu.VMEM((2, CHF, 8 * D), f32),
        pltpu.VMEM((2 * CHF, D), f32),
        pltpu.VMEM_SHARED((N_PAD, D), f32),
        pltpu.SemaphoreType.DMA((2,)),
        pltpu.SemaphoreType.DMA((2,)),
    ],
    compiler_params=_sc_params,
)
def _sc_pass(table_hbm, src3_hbm, dst3_hbm, wa_hbm, wb_hbm, out_hbm,
             sidx_v, didx_v, xs_v, wa_v, wb_v, msg_v, acc_sh, lsem, ssem):
    cid = lax.axis_index("c")
    sid = lax.axis_index("s")
    wid = sid * NC + cid
    pltpu.sync_copy(src3_hbm.at[wid], sidx_v)
    pltpu.sync_copy(dst3_hbm.at[wid], didx_v)

    nch = jnp.where(wid == 0, MAXCH, BASE_CH)

    def issue(j, p):
        eoff = jnp.where(j < BASE_CH,
                         wid * EPT2 + j * CHF,
                         NW * EPT2 + (j - BASE_CH) * CHF)
        pltpu.async_copy(wa_hbm.at[pl.ds(eoff, CHF)], wa_v.at[p], lsem.at[p])
        pltpu.async_copy(wb_hbm.at[pl.ds(eoff, CHF)], wb_v.at[p], lsem.at[p])
        pltpu.async_copy(table_hbm.at[sidx_v.at[j]], xs_v.at[p], lsem.at[p])

    issue(0, 0)

    def zbody(i, carry):
        msg_v[i, :] = jnp.zeros((D,), f32)
        return carry

    lax.fori_loop(0, CHF, zbody, 0, unroll=False)

    def zcopy(i, carry):
        pltpu.sync_copy(msg_v.at[pl.ds(0, CHF)],
                        acc_sh.at[pl.ds(sid * RPT + i * CHF, CHF)])
        return carry

    lax.fori_loop(0, RPT // CHF, zcopy, 0, unroll=False)
    plsc.subcore_barrier()

    def body(j, carry):
        p = lax.rem(j, 2)

        @pl.when(j + 1 < nch)
        def _():
            issue(j + 1, 1 - p)

        pltpu.make_async_copy(
            wa_hbm.at[pl.ds(0, CHF)], wa_v.at[p], lsem.at[p]).wait()
        pltpu.make_async_copy(
            wb_hbm.at[pl.ds(0, CHF)], wb_v.at[p], lsem.at[p]).wait()
        pltpu.make_async_copy(
            table_hbm.at[pl.ds(0, CHF)], xs_v.at[p], lsem.at[p]).wait()

        @pl.when(j >= 2)
        def _():
            pltpu.make_async_copy(
                msg_v.at[pl.ds(p * CHF, CHF)], acc_sh.at[didx_v.at[j - 2]],
                ssem.at[p]).wait()

        def edge(e, carry2):
            xsrow = xs_v[p, e, :]
            acc = xsrow[0] * wa_v[p, e, 0:D]
            for i in range(1, 8):
                acc = acc + xsrow[i] * wa_v[p, e, i * D:(i + 1) * D]
            for i in range(8):
                acc = acc + xsrow[8 + i] * wb_v[p, e, i * D:(i + 1) * D]
            msg_v[p * CHF + e, :] = acc
            return carry2

        lax.fori_loop(0, CHF, edge, 0, unroll=False)
        pltpu.async_copy(msg_v.at[pl.ds(p * CHF, CHF)],
                         acc_sh.at[didx_v.at[j]], ssem.at[p], add=True)
        return carry

    lax.fori_loop(0, nch, body, 0, unroll=False)

    def drain(p, carry):
        pltpu.make_async_copy(
            msg_v.at[pl.ds(p * CHF, CHF)], acc_sh.at[didx_v.at[0]],
            ssem.at[p]).wait()
        return carry

    lax.fori_loop(0, 2, drain, 0, unroll=False)
    plsc.subcore_barrier()
    pltpu.sync_copy(acc_sh.at[pl.ds(sid * RPT, RPT)],
                    out_hbm.at[cid, pl.ds(sid * RPT, RPT)])


BE = 1600
GE = E // BE


def _wprep_body(ea_ref, W1m_ref, b1m_ref, W2m_ref, b2m_ref,
                W1d_ref, b1d_ref, W2d_ref, b2d_ref,
                wam_ref, wbm_ref, wad_ref, wbd_ref):
    ea = ea_ref[...]
    hm = jnp.maximum(
        jnp.dot(ea, W1m_ref[...], preferred_element_type=f32)
        + b1m_ref[...], 0.0)
    wm = jnp.dot(hm, W2m_ref[...], preferred_element_type=f32) + b2m_ref[...]
    wam_ref[...] = wm[:, :8 * D]
    wbm_ref[...] = wm[:, 8 * D:]
    hd = jnp.maximum(
        jnp.dot(ea, W1d_ref[...], preferred_element_type=f32)
        + b1d_ref[...], 0.0)
    wd = jnp.dot(hd, W2d_ref[...], preferred_element_type=f32) + b2d_ref[...]
    wad_ref[...] = wd[:, :8 * D]
    wbd_ref[...] = wd[:, 8 * D:]


_wprep_call = pl.pallas_call(
    _wprep_body,
    grid=(GE,),
    in_specs=[
        pl.BlockSpec((BE, DE), lambda i: (i, 0)),
        pl.BlockSpec((DE, HID), lambda i: (0, 0)),
        pl.BlockSpec((1, HID), lambda i: (0, 0)),
        pl.BlockSpec((HID, D * D), lambda i: (0, 0)),
        pl.BlockSpec((1, D * D), lambda i: (0, 0)),
        pl.BlockSpec((DE, HID), lambda i: (0, 0)),
        pl.BlockSpec((1, HID), lambda i: (0, 0)),
        pl.BlockSpec((HID, D * D), lambda i: (0, 0)),
        pl.BlockSpec((1, D * D), lambda i: (0, 0)),
    ],
    out_specs=[
        pl.BlockSpec((BE, 8 * D), lambda i: (i, 0)),
        pl.BlockSpec((BE, 8 * D), lambda i: (i, 0)),
        pl.BlockSpec((BE, 8 * D), lambda i: (i, 0)),
        pl.BlockSpec((BE, 8 * D), lambda i: (i, 0)),
    ],
    out_shape=[
        jax.ShapeDtypeStruct((E, 8 * D), f32),
        jax.ShapeDtypeStruct((E, 8 * D), f32),
        jax.ShapeDtypeStruct((E, 8 * D), f32),
        jax.ShapeDtypeStruct((E, 8 * D), f32),
    ],
)


def _inv_body(p_ref, out_ref):
    c = p_ref[0] + p_ref[1]
    out_ref[...] = 1.0 / jnp.maximum(c, 1.0)


_inv_call = pl.pallas_call(
    _inv_body,
    grid=(1,),
    in_specs=[pl.BlockSpec((NC, N, D), lambda i: (0, 0, 0))],
    out_specs=pl.BlockSpec((N, D), lambda i: (0, 0)),
    out_shape=jax.ShapeDtypeStruct((N, D), f32),
)


def _combine_body(p_ref, inv_ref, cur_ref, root_ref, bias_ref, out_ref):
    aggr = (p_ref[0] + p_ref[1]) * inv_ref[...]
    out_ref[...] = aggr + jnp.dot(
        cur_ref[...], root_ref[...], preferred_element_type=f32) + bias_ref[...]


_combine_call = pl.pallas_call(
    _combine_body,
    grid=(1,),
    in_specs=[
        pl.BlockSpec((NC, N, D), lambda i: (0, 0, 0)),
        pl.BlockSpec((N, D), lambda i: (0, 0)),
        pl.BlockSpec((N, D), lambda i: (0, 0)),
        pl.BlockSpec((D, D), lambda i: (0, 0)),
        pl.BlockSpec((1, D), lambda i: (0, 0)),
    ],
    out_specs=pl.BlockSpec((N, D), lambda i: (0, 0)),
    out_shape=jax.ShapeDtypeStruct((N, D), f32),
)


def _gru_body(p_ref, inv_ref, m1_ref, h_ref, root_ref, bias_ref, wihT_ref,
              whhT_ref, bih_ref, bhh_ref, out_ref):
    m1 = m1_ref[...]
    h = h_ref[...]
    m2 = ((p_ref[0] + p_ref[1]) * inv_ref[...]
          + jnp.dot(m1, root_ref[...], preferred_element_type=f32)
          + bias_ref[...])
    gi = jnp.dot(m2, wihT_ref[...], preferred_element_type=f32) + bih_ref[...]
    gh = jnp.dot(h, whhT_ref[...], preferred_element_type=f32) + bhh_ref[...]
    r = jax.nn.sigmoid(gi[:, :D] + gh[:, :D])
    z = jax.nn.sigmoid(gi[:, D:2 * D] + gh[:, D:2 * D])
    n_ = jnp.tanh(gi[:, 2 * D:] + r * gh[:, 2 * D:])
    out_ref[...] = (1.0 - z) * n_ + z * h


_gru_call = pl.pallas_call(
    _gru_body,
    grid=(1,),
    in_specs=[
        pl.BlockSpec((NC, N, D), lambda i: (0, 0, 0)),
        pl.BlockSpec((N, D), lambda i: (0, 0)),
        pl.BlockSpec((N, D), lambda i: (0, 0)),
        pl.BlockSpec((N, D), lambda i: (0, 0)),
        pl.BlockSpec((D, D), lambda i: (0, 0)),
        pl.BlockSpec((1, D), lambda i: (0, 0)),
        pl.BlockSpec((D, 3 * D), lambda i: (0, 0)),
        pl.BlockSpec((D, 3 * D), lambda i: (0, 0)),
        pl.BlockSpec((1, 3 * D), lambda i: (0, 0)),
        pl.BlockSpec((1, 3 * D), lambda i: (0, 0)),
    ],
    out_specs=pl.BlockSpec((N, D), lambda i: (0, 0)),
    out_shape=jax.ShapeDtypeStruct((N, D), f32),
)


def _head_body(cur_ref, linW_ref, linb_ref, g_ref, b_ref, scW_ref, scb_ref,
               out_ref):
    cur = cur_ref[...]
    y = jnp.dot(cur, linW_ref[...], preferred_element_type=f32) + linb_ref[...]
    mean = jnp.mean(y, axis=0, keepdims=True)
    var = jnp.mean((y - mean) ** 2, axis=0, keepdims=True)
    yn = (y - mean) / jnp.sqrt(var + 1e-5) * g_ref[...] + b_ref[...]
    out_ref[...] = jnp.maximum(yn, 0.0) + jnp.dot(
        cur, scW_ref[...], preferred_element_type=f32) + scb_ref[...]


_head_call = pl.pallas_call(
    _head_body,
    out_shape=jax.ShapeDtypeStruct((N, DOUT), f32),
)


def kernel(x, edge_index, edge_attr, batch, mp_W1, mp_b1, mp_W2, mp_b2,
           mp_root, mp_bias, dmp_W1, dmp_b1, dmp_W2, dmp_b2, dmp_root,
           dmp_bias, gru_w_ih, gru_w_hh, gru_b_ih, gru_b_hh, lin_W, lin_b,
           bn_gamma, bn_beta, sc_W, sc_b):
    src = edge_index[0]
    dst = edge_index[1]

    def chunk3(idx):
        chunks = idx.reshape(E // CHF, CHF)
        main = chunks[:NW * BASE_CH].reshape(NW, BASE_CH, CHF)
        extras = jnp.zeros((NW, EXTRA, CHF), jnp.int32)
        extras = extras.at[0].set(chunks[NW * BASE_CH:])
        return jnp.concatenate([main, extras], axis=1)

    src3 = chunk3(src)
    dst3 = chunk3(dst)

    mp_b1r = mp_b1.reshape(1, HID)
    mp_b2r = mp_b2.reshape(1, D * D)
    mp_biasr = mp_bias.reshape(1, D)
    dmp_b1r = dmp_b1.reshape(1, HID)
    dmp_b2r = dmp_b2.reshape(1, D * D)
    dmp_biasr = dmp_bias.reshape(1, D)
    wihT = gru_w_ih.T
    whhT = gru_w_hh.T
    bihr = gru_b_ih.reshape(1, 3 * D)
    bhhr = gru_b_hh.reshape(1, 3 * D)
    linbr = lin_b.reshape(1, DOUT)
    gammar = bn_gamma.reshape(1, DOUT)
    betar = bn_beta.reshape(1, DOUT)
    scbr = sc_b.reshape(1, DOUT)

    wam, wbm, wad, wbd = _wprep_call(
        edge_attr, mp_W1, mp_b1r, mp_W2, mp_b2r,
        dmp_W1, dmp_b1r, dmp_W2, dmp_b2r)

    cnt_p = _sc_count(dst3)
    inv = _inv_call(cnt_p)

    h = x
    cur = x
    for _ in range(3):
        p1 = _sc_pass(cur, src3, dst3, wam, wbm)
        m1 = _combine_call(p1, inv, cur, mp_root, mp_biasr)
        p2 = _sc_pass(m1, src3, dst3, wad, wbd)
        h = _gru_call(p2, inv, m1, h, dmp_root, dmp_biasr, wihT, whhT,
                      bihr, bhhr)
        cur = h

    return _head_call(cur, lin_W, linbr, gammar, betar, sc_W, scbr)


# speedup vs baseline: 4.9729x; 1.0044x over previous
import functools

import jax
import jax.numpy as jnp
from jax import lax
from jax.experimental import pallas as pl
from jax.experimental.pallas import tpu as pltpu
from jax.experimental.pallas import tpu_sc as plsc

N = 10000
E = 160000
D = 16
DE = 16
HID = 64
DOUT = 64

NC = 2
NS = 16
NW = NC * NS
EPT = E // NW
CH = 125
NCHUNK = EPT // CH
N_PAD = 10240
RPT = N_PAD // NS

CHF = 128
BASE_CH = 39
EPT2 = BASE_CH * CHF
EXTRA = 2
MAXCH = BASE_CH + EXTRA

_mesh = plsc.VectorSubcoreMesh(
    core_axis_name="c", subcore_axis_name="s", num_cores=NC, num_subcores=NS)

_sc_params = pltpu.CompilerParams(use_tc_tiling_on_sc=False)

f32 = jnp.float32


@functools.partial(
    pl.kernel,
    out_type=jax.ShapeDtypeStruct((NC, N_PAD, D), f32),
    mesh=_mesh,
    scratch_types=[
        pltpu.VMEM((MAXCH, CHF), jnp.int32),
        pltpu.VMEM((CHF, D), f32),
        pltpu.VMEM_SHARED((N_PAD, D), f32),
        pltpu.SemaphoreType.DMA,
    ],
    compiler_params=_sc_params,
)
def _sc_count(dst3_hbm, out_hbm, didx_v, ones_v, acc_sh, sem):
    cid = lax.axis_index("c")
    sid = lax.axis_index("s")
    wid = sid * NC + cid
    pltpu.sync_copy(dst3_hbm.at[wid], didx_v)

    def zbody(i, carry):
        ones_v[i, :] = jnp.zeros((D,), f32)
        return carry

    lax.fori_loop(0, CHF, zbody, 0, unroll=False)

    def zcopy(i, carry):
        pltpu.sync_copy(ones_v, acc_sh.at[pl.ds(sid * RPT + i * CHF, CHF)])
        return carry

    lax.fori_loop(0, RPT // CHF, zcopy, 0, unroll=False)

    def obody(i, carry):
        ones_v[i, :] = jnp.ones((D,), f32)
        return carry

    lax.fori_loop(0, CHF, obody, 0, unroll=False)
    plsc.subcore_barrier()

    nch = jnp.where(wid == 0, MAXCH, BASE_CH)

    def body(j, carry):
        pltpu.sync_copy(ones_v, acc_sh.at[didx_v.at[j]], add=True)
        return carry

    lax.fori_loop(0, nch, body, 0, unroll=False)
    plsc.subcore_barrier()
    pltpu.sync_copy(acc_sh.at[pl.ds(sid * RPT, RPT)],
                    out_hbm.at[cid, pl.ds(sid * RPT, RPT)])


@functools.partial(
    pl.kernel,
    out_type=jax.ShapeDtypeStruct((NC, N_PAD, D), f32),
    mesh=_mesh,
    scratch_types=[
        pltpu.VMEM((MAXCH, CHF), jnp.int32),
        pltpu.VMEM((MAXCH, CHF), jnp.int32),
        pltpu.VMEM((3, CHF, D), f32),
        pltpu.VMEM((3, CHF, 8 * D), f32),
        pltpu.VMEM((3, CHF, 8 * D), f32),
        pltpu.VMEM((2 * CHF, D), f32),
        pltpu.VMEM_SHARED((N_PAD, D), f32),
        pltpu.SemaphoreType.DMA((3,)),
        pltpu.SemaphoreType.DMA((2,)),
    ],
    compiler_params=_sc_params,
)
def _sc_pass(table_hbm, src3_hbm, dst3_hbm, wa_hbm, wb_hbm, out_hbm,
             sidx_v, didx_v, xs_v, wa_v, wb_v, msg_v, acc_sh, lsem, ssem):
    cid = lax.axis_index("c")
    sid = lax.axis_index("s")
    wid = sid * NC + cid
    pltpu.sync_copy(src3_hbm.at[wid], sidx_v)
    pltpu.sync_copy(dst3_hbm.at[wid], didx_v)

    nch = jnp.where(wid == 0, MAXCH, BASE_CH)

    def issue(j, p):
        eoff = jnp.where(j < BASE_CH,
                         wid * EPT2 + j * CHF,
                         NW * EPT2 + (j - BASE_CH) * CHF)
        pltpu.async_copy(wa_hbm.at[pl.ds(eoff, CHF)], wa_v.at[p], lsem.at[p])
        pltpu.async_copy(wb_hbm.at[pl.ds(eoff, CHF)], wb_v.at[p], lsem.at[p])
        pltpu.async_copy(table_hbm.at[sidx_v.at[j]], xs_v.at[p], lsem.at[p])

    issue(0, 0)
    issue(1, 1)

    def zbody(i, carry):
        msg_v[i, :] = jnp.zeros((D,), f32)
        return carry

    lax.fori_loop(0, CHF, zbody, 0, unroll=False)

    def zcopy(i, carry):
        pltpu.sync_copy(msg_v.at[pl.ds(0, CHF)],
                        acc_sh.at[pl.ds(sid * RPT + i * CHF, CHF)])
        return carry

    lax.fori_loop(0, RPT // CHF, zcopy, 0, unroll=False)
    plsc.subcore_barrier()

    def body(j, carry):
        p = lax.rem(j, 3)
        q = lax.rem(j, 2)

        @pl.when(j + 2 < nch)
        def _():
            issue(j + 2, lax.rem(j + 2, 3))

        pltpu.make_async_copy(
            wa_hbm.at[pl.ds(0, CHF)], wa_v.at[p], lsem.at[p]).wait()
        pltpu.make_async_copy(
            wb_hbm.at[pl.ds(0, CHF)], wb_v.at[p], lsem.at[p]).wait()
        pltpu.make_async_copy(
            table_hbm.at[pl.ds(0, CHF)], xs_v.at[p], lsem.at[p]).wait()

        @pl.when(j >= 2)
        def _():
            pltpu.make_async_copy(
                msg_v.at[pl.ds(q * CHF, CHF)], acc_sh.at[didx_v.at[j - 2]],
                ssem.at[q]).wait()

        def edge(e, carry2):
            xsrow = xs_v[p, e, :]
            acc = xsrow[0] * wa_v[p, e, 0:D]
            for i in range(1, 8):
                acc = acc + xsrow[i] * wa_v[p, e, i * D:(i + 1) * D]
            for i in range(8):
                acc = acc + xsrow[8 + i] * wb_v[p, e, i * D:(i + 1) * D]
            msg_v[q * CHF + e, :] = acc
            return carry2

        lax.fori_loop(0, CHF, edge, 0, unroll=False)
        pltpu.async_copy(msg_v.at[pl.ds(q * CHF, CHF)],
                         acc_sh.at[didx_v.at[j]], ssem.at[q], add=True)
        return carry

    lax.fori_loop(0, nch, body, 0, unroll=False)

    def drain(p, carry):
        pltpu.make_async_copy(
            msg_v.at[pl.ds(p * CHF, CHF)], acc_sh.at[didx_v.at[0]],
            ssem.at[p]).wait()
        return carry

    lax.fori_loop(0, 2, drain, 0, unroll=False)
    plsc.subcore_barrier()
    pltpu.sync_copy(acc_sh.at[pl.ds(sid * RPT, RPT)],
                    out_hbm.at[cid, pl.ds(sid * RPT, RPT)])


BE = 1600
GE = E // BE


def _wprep_body(ea_ref, W1m_ref, b1m_ref, W2m_ref, b2m_ref,
                W1d_ref, b1d_ref, W2d_ref, b2d_ref,
                wam_ref, wbm_ref, wad_ref, wbd_ref):
    ea = ea_ref[...]
    hm = jnp.maximum(
        jnp.dot(ea, W1m_ref[...], preferred_element_type=f32)
        + b1m_ref[...], 0.0)
    wm = jnp.dot(hm, W2m_ref[...], preferred_element_type=f32) + b2m_ref[...]
    wam_ref[...] = wm[:, :8 * D]
    wbm_ref[...] = wm[:, 8 * D:]
    hd = jnp.maximum(
        jnp.dot(ea, W1d_ref[...], preferred_element_type=f32)
        + b1d_ref[...], 0.0)
    wd = jnp.dot(hd, W2d_ref[...], preferred_element_type=f32) + b2d_ref[...]
    wad_ref[...] = wd[:, :8 * D]
    wbd_ref[...] = wd[:, 8 * D:]


_wprep_call = pl.pallas_call(
    _wprep_body,
    grid=(GE,),
    in_specs=[
        pl.BlockSpec((BE, DE), lambda i: (i, 0)),
        pl.BlockSpec((DE, HID), lambda i: (0, 0)),
        pl.BlockSpec((1, HID), lambda i: (0, 0)),
        pl.BlockSpec((HID, D * D), lambda i: (0, 0)),
        pl.BlockSpec((1, D * D), lambda i: (0, 0)),
        pl.BlockSpec((DE, HID), lambda i: (0, 0)),
        pl.BlockSpec((1, HID), lambda i: (0, 0)),
        pl.BlockSpec((HID, D * D), lambda i: (0, 0)),
        pl.BlockSpec((1, D * D), lambda i: (0, 0)),
    ],
    out_specs=[
        pl.BlockSpec((BE, 8 * D), lambda i: (i, 0)),
        pl.BlockSpec((BE, 8 * D), lambda i: (i, 0)),
        pl.BlockSpec((BE, 8 * D), lambda i: (i, 0)),
        pl.BlockSpec((BE, 8 * D), lambda i: (i, 0)),
    ],
    out_shape=[
        jax.ShapeDtypeStruct((E, 8 * D), f32),
        jax.ShapeDtypeStruct((E, 8 * D), f32),
        jax.ShapeDtypeStruct((E, 8 * D), f32),
        jax.ShapeDtypeStruct((E, 8 * D), f32),
    ],
)


def _inv_body(p_ref, out_ref):
    c = p_ref[0] + p_ref[1]
    out_ref[...] = 1.0 / jnp.maximum(c, 1.0)


_inv_call = pl.pallas_call(
    _inv_body,
    grid=(1,),
    in_specs=[pl.BlockSpec((NC, N, D), lambda i: (0, 0, 0))],
    out_specs=pl.BlockSpec((N, D), lambda i: (0, 0)),
    out_shape=jax.ShapeDtypeStruct((N, D), f32),
)


def _combine_body(p_ref, inv_ref, cur_ref, root_ref, bias_ref, out_ref):
    aggr = (p_ref[0] + p_ref[1]) * inv_ref[...]
    out_ref[...] = aggr + jnp.dot(
        cur_ref[...], root_ref[...], preferred_element_type=f32) + bias_ref[...]


_combine_call = pl.pallas_call(
    _combine_body,
    grid=(1,),
    in_specs=[
        pl.BlockSpec((NC, N, D), lambda i: (0, 0, 0)),
        pl.BlockSpec((N, D), lambda i: (0, 0)),
        pl.BlockSpec((N, D), lambda i: (0, 0)),
        pl.BlockSpec((D, D), lambda i: (0, 0)),
        pl.BlockSpec((1, D), lambda i: (0, 0)),
    ],
    out_specs=pl.BlockSpec((N, D), lambda i: (0, 0)),
    out_shape=jax.ShapeDtypeStruct((N, D), f32),
)


def _gru_body(p_ref, inv_ref, m1_ref, h_ref, root_ref, bias_ref, wihT_ref,
              whhT_ref, bih_ref, bhh_ref, out_ref):
    m1 = m1_ref[...]
    h = h_ref[...]
    m2 = ((p_ref[0] + p_ref[1]) * inv_ref[...]
          + jnp.dot(m1, root_ref[...], preferred_element_type=f32)
          + bias_ref[...])
    gi = jnp.dot(m2, wihT_ref[...], preferred_element_type=f32) + bih_ref[...]
    gh = jnp.dot(h, whhT_ref[...], preferred_element_type=f32) + bhh_ref[...]
    r = jax.nn.sigmoid(gi[:, :D] + gh[:, :D])
    z = jax.nn.sigmoid(gi[:, D:2 * D] + gh[:, D:2 * D])
    n_ = jnp.tanh(gi[:, 2 * D:] + r * gh[:, 2 * D:])
    out_ref[...] = (1.0 - z) * n_ + z * h


_gru_call = pl.pallas_call(
    _gru_body,
    grid=(1,),
    in_specs=[
        pl.BlockSpec((NC, N, D), lambda i: (0, 0, 0)),
        pl.BlockSpec((N, D), lambda i: (0, 0)),
        pl.BlockSpec((N, D), lambda i: (0, 0)),
        pl.BlockSpec((N, D), lambda i: (0, 0)),
        pl.BlockSpec((D, D), lambda i: (0, 0)),
        pl.BlockSpec((1, D), lambda i: (0, 0)),
        pl.BlockSpec((D, 3 * D), lambda i: (0, 0)),
        pl.BlockSpec((D, 3 * D), lambda i: (0, 0)),
        pl.BlockSpec((1, 3 * D), lambda i: (0, 0)),
        pl.BlockSpec((1, 3 * D), lambda i: (0, 0)),
    ],
    out_specs=pl.BlockSpec((N, D), lambda i: (0, 0)),
    out_shape=jax.ShapeDtypeStruct((N, D), f32),
)


def _head_body(cur_ref, linW_ref, linb_ref, g_ref, b_ref, scW_ref, scb_ref,
               out_ref):
    cur = cur_ref[...]
    y = jnp.dot(cur, linW_ref[...], preferred_element_type=f32) + linb_ref[...]
    mean = jnp.mean(y, axis=0, keepdims=True)
    var = jnp.mean((y - mean) ** 2, axis=0, keepdims=True)
    yn = (y - mean) / jnp.sqrt(var + 1e-5) * g_ref[...] + b_ref[...]
    out_ref[...] = jnp.maximum(yn, 0.0) + jnp.dot(
        cur, scW_ref[...], preferred_element_type=f32) + scb_ref[...]


_head_call = pl.pallas_call(
    _head_body,
    out_shape=jax.ShapeDtypeStruct((N, DOUT), f32),
)


def kernel(x, edge_index, edge_attr, batch, mp_W1, mp_b1, mp_W2, mp_b2,
           mp_root, mp_bias, dmp_W1, dmp_b1, dmp_W2, dmp_b2, dmp_root,
           dmp_bias, gru_w_ih, gru_w_hh, gru_b_ih, gru_b_hh, lin_W, lin_b,
           bn_gamma, bn_beta, sc_W, sc_b):
    src = edge_index[0]
    dst = edge_index[1]

    def chunk3(idx):
        chunks = idx.reshape(E // CHF, CHF)
        main = chunks[:NW * BASE_CH].reshape(NW, BASE_CH, CHF)
        extras = jnp.zeros((NW, EXTRA, CHF), jnp.int32)
        extras = extras.at[0].set(chunks[NW * BASE_CH:])
        return jnp.concatenate([main, extras], axis=1)

    src3 = chunk3(src)
    dst3 = chunk3(dst)

    mp_b1r = mp_b1.reshape(1, HID)
    mp_b2r = mp_b2.reshape(1, D * D)
    mp_biasr = mp_bias.reshape(1, D)
    dmp_b1r = dmp_b1.reshape(1, HID)
    dmp_b2r = dmp_b2.reshape(1, D * D)
    dmp_biasr = dmp_bias.reshape(1, D)
    wihT = gru_w_ih.T
    whhT = gru_w_hh.T
    bihr = gru_b_ih.reshape(1, 3 * D)
    bhhr = gru_b_hh.reshape(1, 3 * D)
    linbr = lin_b.reshape(1, DOUT)
    gammar = bn_gamma.reshape(1, DOUT)
    betar = bn_beta.reshape(1, DOUT)
    scbr = sc_b.reshape(1, DOUT)

    wam, wbm, wad, wbd = _wprep_call(
        edge_attr, mp_W1, mp_b1r, mp_W2, mp_b2r,
        dmp_W1, dmp_b1r, dmp_W2, dmp_b2r)

    cnt_p = _sc_count(dst3)
    inv = _inv_call(cnt_p)

    h = x
    cur = x
    for _ in range(3):
        p1 = _sc_pass(cur, src3, dst3, wam, wbm)
        m1 = _combine_call(p1, inv, cur, mp_root, mp_biasr)
        p2 = _sc_pass(m1, src3, dst3, wad, wbd)
        h = _gru_call(p2, inv, m1, h, dmp_root, dmp_biasr, wihT, whhT,
                      bihr, bhhr)
        cur = h

    return _head_call(cur, lin_W, linbr, gammar, betar, sc_W, scbr)
